# baseline (speedup 1.0000x reference)
import jax
import jax.numpy as jnp
from jax.experimental import pallas as pl
from jax.experimental.pallas import tpu as pltpu

N, D_IN, D_OUT, TOPK = 8192, 1024, 1024, 8
BLOCK_ROWS = 256
SUB = 8


def _body(x_ref, w_ref, bias_ref, o_ref, y_ref, th_ref):
    x = x_ref[...]
    y_ref[...] = jax.lax.dot_general(
        x, w_ref[...], (((1,), (0,)), ((), ())),
        preferred_element_type=jnp.float32,
    ) + bias_ref[...]

    def sub_threshold(i, _):
        t = y_ref[pl.ds(i * SUB, SUB), :D_OUT]
        for _ in range(TOPK - 1):
            m = jnp.max(t, axis=1, keepdims=True)
            t = jnp.where(t >= m, -jnp.inf, t)
        th_ref[pl.ds(i * SUB, SUB), :] = jnp.max(t, axis=1, keepdims=True)
        return 0

    jax.lax.fori_loop(0, BLOCK_ROWS // SUB, sub_threshold, 0)
    logits = y_ref[:, :D_OUT]
    orig = y_ref[:, D_OUT:]
    mask = (logits >= th_ref[...]).astype(jnp.float32)
    o_ref[...] = orig * mask


@jax.jit
def kernel(x, Wr, br, W, b):
    w_cat = jnp.concatenate([Wr.T, W.T], axis=1)
    bias_cat = jnp.concatenate([br, b]).reshape(1, 2 * D_OUT)
    grid = (N // BLOCK_ROWS,)
    return pl.pallas_call(
        _body,
        grid=grid,
        in_specs=[
            pl.BlockSpec((BLOCK_ROWS, D_IN), lambda i: (i, 0)),
            pl.BlockSpec((D_IN, 2 * D_OUT), lambda i: (0, 0)),
            pl.BlockSpec((1, 2 * D_OUT), lambda i: (0, 0)),
        ],
        out_specs=pl.BlockSpec((BLOCK_ROWS, D_OUT), lambda i: (i, 0)),
        out_shape=jax.ShapeDtypeStruct((N, D_OUT), jnp.float32),
        scratch_shapes=[
            pltpu.VMEM((BLOCK_ROWS, 2 * D_OUT), jnp.float32),
            pltpu.VMEM((BLOCK_ROWS, 1), jnp.float32),
        ],
    )(x, w_cat, bias_cat)


# speedup vs baseline: 8.9484x; 8.9484x over previous
import jax
import jax.numpy as jnp
from jax.experimental import pallas as pl

N, D_IN, D_OUT, TOPK = 8192, 1024, 1024, 8
BLOCK_ROWS = 256
GROUP = 64


def _body(x_ref, w_ref, bias_ref, o_ref):
    x = x_ref[...]
    y = jax.lax.dot_general(
        x, w_ref[...], (((1,), (0,)), ((), ())),
        preferred_element_type=jnp.float32,
    ) + bias_ref[...]
    for g in range(BLOCK_ROWS // GROUP):
        logits = y[g * GROUP:(g + 1) * GROUP, :D_OUT]
        orig = y[g * GROUP:(g + 1) * GROUP, D_OUT:]
        t = logits
        for _ in range(TOPK - 1):
            m = jnp.max(t, axis=1, keepdims=True)
            t = jnp.where(t >= m, -jnp.inf, t)
        thresh = jnp.max(t, axis=1, keepdims=True)
        mask = (logits >= thresh).astype(jnp.float32)
        o_ref[g * GROUP:(g + 1) * GROUP, :] = orig * mask


@jax.jit
def kernel(x, Wr, br, W, b):
    w_cat = jnp.concatenate([Wr.T, W.T], axis=1)
    bias_cat = jnp.concatenate([br, b]).reshape(1, 2 * D_OUT)
    grid = (N // BLOCK_ROWS,)
    return pl.pallas_call(
        _body,
        grid=grid,
        in_specs=[
            pl.BlockSpec((BLOCK_ROWS, D_IN), lambda i: (i, 0)),
            pl.BlockSpec((D_IN, 2 * D_OUT), lambda i: (0, 0)),
            pl.BlockSpec((1, 2 * D_OUT), lambda i: (0, 0)),
        ],
        out_specs=pl.BlockSpec((BLOCK_ROWS, D_OUT), lambda i: (i, 0)),
        out_shape=jax.ShapeDtypeStruct((N, D_OUT), jnp.float32),
    )(x, w_cat, bias_cat)


# speedup vs baseline: 9.9163x; 1.1082x over previous
import jax
import jax.numpy as jnp
from jax.experimental import pallas as pl

N, D_IN, D_OUT, TOPK = 8192, 1024, 1024, 8
BLOCK_ROWS = 256


def _body(x_ref, wrt_ref, br_ref, wt_ref, b_ref, o_ref):
    x = x_ref[...]
    logits = jax.lax.dot_general(
        x, wrt_ref[...], (((1,), (0,)), ((), ())),
        preferred_element_type=jnp.float32,
    ) + br_ref[...]
    t = logits
    for _ in range(TOPK - 1):
        m = jnp.max(t, axis=1, keepdims=True)
        t = jnp.where(t >= m, -jnp.inf, t)
    thresh = jnp.max(t, axis=1, keepdims=True)
    mask = (logits >= thresh).astype(jnp.float32)
    orig = jax.lax.dot_general(
        x.astype(jnp.bfloat16), wt_ref[...], (((1,), (0,)), ((), ())),
        preferred_element_type=jnp.float32,
    ) + b_ref[...]
    o_ref[...] = orig * mask


@jax.jit
def kernel(x, Wr, br, W, b):
    wrt = Wr.T
    wt = W.T.astype(jnp.bfloat16)
    br2 = br.reshape(1, D_OUT)
    b2 = b.reshape(1, D_OUT)
    grid = (N // BLOCK_ROWS,)
    return pl.pallas_call(
        _body,
        grid=grid,
        in_specs=[
            pl.BlockSpec((BLOCK_ROWS, D_IN), lambda i: (i, 0)),
            pl.BlockSpec((D_IN, D_OUT), lambda i: (0, 0)),
            pl.BlockSpec((1, D_OUT), lambda i: (0, 0)),
            pl.BlockSpec((D_IN, D_OUT), lambda i: (0, 0)),
            pl.BlockSpec((1, D_OUT), lambda i: (0, 0)),
        ],
        out_specs=pl.BlockSpec((BLOCK_ROWS, D_OUT), lambda i: (i, 0)),
        out_shape=jax.ShapeDtypeStruct((N, D_OUT), jnp.float32),
    )(x, wrt, br2, wt, b2)


# speedup vs baseline: 10.0140x; 1.0099x over previous
import jax
import jax.numpy as jnp
from jax.experimental import pallas as pl
from jax.experimental.pallas import tpu as pltpu

N, D_IN, D_OUT, TOPK = 8192, 1024, 1024, 8
BLOCK_ROWS = 512


def _body(x_ref, wrt_ref, br_ref, wt_ref, b_ref, o_ref):
    x = x_ref[...]
    logits = jax.lax.dot_general(
        x, wrt_ref[...], (((1,), (0,)), ((), ())),
        preferred_element_type=jnp.float32,
    ) + br_ref[...]
    t = logits
    for _ in range(TOPK - 1):
        m = jnp.max(t, axis=1, keepdims=True)
        t = jnp.where(t >= m, -jnp.inf, t)
    thresh = jnp.max(t, axis=1, keepdims=True)
    mask = (logits >= thresh).astype(jnp.float32)
    orig = jax.lax.dot_general(
        x.astype(jnp.bfloat16), wt_ref[...], (((1,), (0,)), ((), ())),
        preferred_element_type=jnp.float32,
    ) + b_ref[...]
    o_ref[...] = orig * mask


@jax.jit
def kernel(x, Wr, br, W, b):
    wrt = Wr.T
    wt = W.T.astype(jnp.bfloat16)
    br2 = br.reshape(1, D_OUT)
    b2 = b.reshape(1, D_OUT)
    grid = (N // BLOCK_ROWS,)
    return pl.pallas_call(
        _body,
        grid=grid,
        in_specs=[
            pl.BlockSpec((BLOCK_ROWS, D_IN), lambda i: (i, 0)),
            pl.BlockSpec((D_IN, D_OUT), lambda i: (0, 0)),
            pl.BlockSpec((1, D_OUT), lambda i: (0, 0)),
            pl.BlockSpec((D_IN, D_OUT), lambda i: (0, 0)),
            pl.BlockSpec((1, D_OUT), lambda i: (0, 0)),
        ],
        out_specs=pl.BlockSpec((BLOCK_ROWS, D_OUT), lambda i: (i, 0)),
        out_shape=jax.ShapeDtypeStruct((N, D_OUT), jnp.float32),
        compiler_params=pltpu.CompilerParams(
            dimension_semantics=("parallel",),
        ),
    )(x, wrt, br2, wt, b2)
